# baseline (speedup 1.0000x reference)
import functools

import jax
import jax.numpy as jnp
from jax import lax
from jax.experimental import pallas as pl
from jax.experimental.pallas import tpu as pltpu
from jax.experimental.pallas import tpu_sc as plsc

N_ROWS = 8192
D = 4096
TOPK = 16
BLOCK_ROWS = 512
N_BLOCKS = N_ROWS // BLOCK_ROWS
INV_TEMP = 10.0


def _sims_topk_body(an_ref, cb_ref, w_ref, i_ref, sims_scr):
    pid = pl.program_id(0)
    blk = cb_ref[...]
    normsq = jnp.sum(blk * blk, axis=1, keepdims=True)
    norm = jnp.maximum(jnp.sqrt(normsq), 1e-12)
    bn = blk / norm
    a = an_ref[...]
    a_norm = jnp.maximum(jnp.sqrt(jnp.sum(a * a)), 1e-12)
    an = a / a_norm
    bn_r = bn.astype(jnp.bfloat16).astype(jnp.float32)
    an_r = an.astype(jnp.bfloat16).astype(jnp.float32)
    s = lax.dot_general(
        bn_r, an_r, (((1,), (1,)), ((), ())), precision=lax.Precision.HIGHEST
    )
    sims_scr[pid, :] = s[:, 0]

    @pl.when(pid == N_BLOCKS - 1)
    def _finalize():
        sims = sims_scr[...]
        ids = (
            lax.broadcasted_iota(jnp.int32, sims.shape, 0) * BLOCK_ROWS
            + lax.broadcasted_iota(jnp.int32, sims.shape, 1)
        )

        def step(k, carry):
            sims_c, vals, idxs = carry
            m = jnp.max(sims_c)
            pick = jnp.min(jnp.where(sims_c == m, ids, jnp.int32(2**30)))
            lane = lax.broadcasted_iota(jnp.int32, (1, TOPK), 1)
            vals = jnp.where(lane == k, m, vals)
            idxs = jnp.where(lane == k, pick, idxs)
            sims_c = jnp.where(ids == pick, -jnp.inf, sims_c)
            return sims_c, vals, idxs

        init = (
            sims,
            jnp.zeros((1, TOPK), jnp.float32),
            jnp.zeros((1, TOPK), jnp.int32),
        )
        _, vals, idxs = lax.fori_loop(0, TOPK, step, init)
        z = vals * INV_TEMP
        z = z - jnp.max(z)
        e = jnp.exp(z)
        w_ref[...] = e / jnp.sum(e)
        i_ref[...] = idxs


def _sims_topk(cb, an):
    return pl.pallas_call(
        _sims_topk_body,
        grid=(N_BLOCKS,),
        in_specs=[
            pl.BlockSpec((1, D), lambda i: (0, 0)),
            pl.BlockSpec((BLOCK_ROWS, D), lambda i: (i, 0)),
        ],
        out_specs=[
            pl.BlockSpec((1, TOPK), lambda i: (0, 0)),
            pl.BlockSpec((1, TOPK), lambda i: (0, 0)),
        ],
        out_shape=[
            jax.ShapeDtypeStruct((1, TOPK), jnp.float32),
            jax.ShapeDtypeStruct((1, TOPK), jnp.int32),
        ],
        scratch_shapes=[pltpu.VMEM((N_BLOCKS, BLOCK_ROWS), jnp.float32)],
    )(an, cb)


def _gather_combine(cb, idxs, weights):
    mesh = plsc.VectorSubcoreMesh(core_axis_name="c", subcore_axis_name="s")

    @functools.partial(
        pl.kernel,
        out_type=jax.ShapeDtypeStruct((D,), jnp.float32),
        mesh=mesh,
        scratch_types=[
            pltpu.VMEM((TOPK,), jnp.int32),
            pltpu.VMEM((TOPK,), jnp.float32),
            pltpu.VMEM((TOPK, D), jnp.float32),
            pltpu.VMEM((D,), jnp.float32),
            pltpu.SemaphoreType.DMA,
        ],
        compiler_params=pltpu.CompilerParams(needs_layout_passes=False),
    )
    def k(cb_hbm, idx_hbm, w_hbm, out_hbm, idx_v, w_v, rows_v, acc_v, sem):
        cid = lax.axis_index("c")
        sid = lax.axis_index("s")

        @pl.when((cid == 0) & (sid == 0))
        def _():
            pltpu.sync_copy(idx_hbm, idx_v)
            pltpu.sync_copy(w_hbm, w_v)
            pltpu.async_copy(cb_hbm.at[idx_v], rows_v, sem).wait()
            w_vec = w_v[...]
            lane = lax.iota(jnp.int32, 16)
            wts = [
                jnp.sum(jnp.where(lane == i, w_vec, 0.0))
                for i in range(TOPK)
            ]

            def chunk(ci, _):
                acc = jnp.zeros((16,), jnp.float32)
                for i in range(TOPK):
                    acc = acc + rows_v[i, pl.ds(ci * 16, 16)] * wts[i]
                acc_v[pl.ds(ci * 16, 16)] = acc
                return 0

            lax.fori_loop(0, D // 16, chunk, 0)
            pltpu.sync_copy(acc_v, out_hbm)

    return k(cb, idxs, weights)


def kernel(codebook, anchor_noise):
    cb = codebook.reshape(N_ROWS, D)
    an = anchor_noise.reshape(1, D)
    weights, idxs = _sims_topk(cb, an)
    out = _gather_combine(cb, idxs.reshape(TOPK), weights.reshape(TOPK))
    return out.reshape(1, 4, 32, 32)


# speedup vs baseline: 1.1914x; 1.1914x over previous
import jax
import jax.numpy as jnp
from jax import lax
from jax.experimental import pallas as pl
from jax.experimental.pallas import tpu as pltpu

N_ROWS = 8192
D = 4096
TOPK = 16
BLOCK_ROWS = 512
N_BLOCKS = N_ROWS // BLOCK_ROWS
INV_TEMP = 10.0


def _body(an_ref, cb_ref, cb_any, out_ref, sims_scr, idx_smem, rows_scr, sem):
    pid = pl.program_id(0)
    blk = cb_ref[...]
    normsq = jnp.sum(blk * blk, axis=1)
    rnorm = 1.0 / jnp.maximum(jnp.sqrt(normsq), 1e-12)
    bn = blk * rnorm[:, None]
    a = an_ref[...]
    a_rnorm = 1.0 / jnp.maximum(jnp.sqrt(jnp.sum(a * a)), 1e-12)
    an = a * a_rnorm
    bn_r = bn.astype(jnp.bfloat16).astype(jnp.float32)
    an_r = an.astype(jnp.bfloat16).astype(jnp.float32)
    s = jnp.sum(bn_r * an_r, axis=1)
    sims_scr[pid, :] = s

    @pl.when(pid == N_BLOCKS - 1)
    def _finalize():
        sims = sims_scr[...]
        ids = (
            lax.broadcasted_iota(jnp.int32, sims.shape, 0) * BLOCK_ROWS
            + lax.broadcasted_iota(jnp.int32, sims.shape, 1)
        )

        def step(k, carry):
            sims_c, vals = carry
            m = jnp.max(sims_c)
            pick = jnp.min(jnp.where(sims_c == m, ids, jnp.int32(2**30)))
            idx_smem[k] = pick
            lane = lax.broadcasted_iota(jnp.int32, (1, TOPK), 1)
            vals = jnp.where(lane == k, m, vals)
            sims_c = jnp.where(ids == pick, -jnp.inf, sims_c)
            return sims_c, vals

        _, vals = lax.fori_loop(
            0, TOPK, step, (sims, jnp.zeros((1, TOPK), jnp.float32))
        )

        copies = [
            pltpu.make_async_copy(
                cb_any.at[pl.ds(idx_smem[k], 1), :],
                rows_scr.at[pl.ds(k, 1), :],
                sem,
            )
            for k in range(TOPK)
        ]
        for c in copies:
            c.start()

        z = vals * INV_TEMP
        z = z - jnp.max(z)
        e = jnp.exp(z)
        w = e / jnp.sum(e)

        for c in copies:
            c.wait()
        rows = rows_scr[...]
        out_ref[...] = jnp.sum(rows * w.reshape(TOPK, 1), axis=0, keepdims=True)


def kernel(codebook, anchor_noise):
    cb = codebook.reshape(N_ROWS, D)
    an = anchor_noise.reshape(1, D)
    out = pl.pallas_call(
        _body,
        grid=(N_BLOCKS,),
        in_specs=[
            pl.BlockSpec((1, D), lambda i: (0, 0)),
            pl.BlockSpec((BLOCK_ROWS, D), lambda i: (i, 0)),
            pl.BlockSpec(memory_space=pl.ANY),
        ],
        out_specs=pl.BlockSpec((1, D), lambda i: (0, 0)),
        out_shape=jax.ShapeDtypeStruct((1, D), jnp.float32),
        scratch_shapes=[
            pltpu.VMEM((N_BLOCKS, BLOCK_ROWS), jnp.float32),
            pltpu.SMEM((TOPK,), jnp.int32),
            pltpu.VMEM((TOPK, D), jnp.float32),
            pltpu.SemaphoreType.DMA,
        ],
    )(an, cb, cb)
    return out.reshape(1, 4, 32, 32)
